# baseline (speedup 1.0000x reference)
import jax
import jax.numpy as jnp
from jax.experimental import pallas as pl

_K = 1024
_D = 64
_BETA = 0.25
_TB = 2048
_BB = 2


def _prep(w_ref, w2_ref, wbf_ref, wneg2_ref, iotaf_ref):
    w0 = w_ref[...]
    w2_ref[...] = jnp.sum(w0 * w0, axis=1)[:, None]
    wbf_ref[...] = w0.astype(jnp.bfloat16)
    wneg2_ref[...] = w0 * -2.0
    iotaf_ref[...] = jax.lax.broadcasted_iota(
        jnp.int32, (_K, 1), 0).astype(jnp.float32)


def _vq_block(lat_ref, wneg2_ref, w2_ref, wbf_ref, iotaf_ref,
              q_ref, idx_ref, acc_ref):
    b = pl.program_id(0)

    @pl.when(b == 0)
    def _():
        acc_ref[...] = jnp.zeros((1, _TB), jnp.float32)

    for i in range(_BB):
        x = lat_ref[i]

        x2 = jnp.sum(x * x, axis=0, keepdims=True)
        sn2 = jax.lax.dot_general(
            wneg2_ref[...], x, (((1,), (0,)), ((), ())),
            preferred_element_type=jnp.float32,
            precision=jax.lax.Precision.DEFAULT)
        dist = (x2 + w2_ref[...]) + sn2

        m = jnp.min(dist, axis=0, keepdims=True)
        cand = jnp.where(dist == m, iotaf_ref[...], jnp.float32(_K))
        idx_f = jnp.min(cand, axis=0, keepdims=True)
        idx = idx_f.astype(jnp.int32)
        idx_ref[i, 0, :] = idx[0, :]

        iota = jax.lax.broadcasted_iota(jnp.int32, (_K, _TB), 0)
        onehot = (iota == idx).astype(jnp.bfloat16)
        q = jax.lax.dot_general(
            wbf_ref[...], onehot, (((0,), (0,)), ((), ())),
            preferred_element_type=jnp.float32)
        q_ref[i] = q

        acc_ref[...] += m


def kernel(latents, weight):
    B, D, T = latents.shape
    nt = T // _TB

    w2, wbf, wneg2, iotaf = pl.pallas_call(
        _prep,
        out_shape=[
            jax.ShapeDtypeStruct((_K, 1), jnp.float32),
            jax.ShapeDtypeStruct((_K, _D), jnp.bfloat16),
            jax.ShapeDtypeStruct((_K, _D), jnp.float32),
            jax.ShapeDtypeStruct((_K, 1), jnp.float32),
        ],
    )(weight)

    q, idx3, acc = pl.pallas_call(
        _vq_block,
        grid=(B // _BB, nt),
        in_specs=[
            pl.BlockSpec((_BB, D, _TB), lambda b, t: (b, 0, t)),
            pl.BlockSpec((_K, _D), lambda b, t: (0, 0)),
            pl.BlockSpec((_K, 1), lambda b, t: (0, 0)),
            pl.BlockSpec((_K, _D), lambda b, t: (0, 0)),
            pl.BlockSpec((_K, 1), lambda b, t: (0, 0)),
        ],
        out_specs=[
            pl.BlockSpec((_BB, D, _TB), lambda b, t: (b, 0, t)),
            pl.BlockSpec((_BB, 1, _TB), lambda b, t: (b, 0, t)),
            pl.BlockSpec((1, _TB), lambda b, t: (0, 0)),
        ],
        out_shape=[
            jax.ShapeDtypeStruct((B, D, T), jnp.float32),
            jax.ShapeDtypeStruct((B, 1, T), jnp.int32),
            jax.ShapeDtypeStruct((1, _TB), jnp.float32),
        ],
    )(latents, wneg2, w2, wbf, iotaf)
    mean_sq = jnp.sum(acc) / (B * T * D)
    loss = mean_sq + _BETA * mean_sq
    return q, loss, idx3.reshape(B, T)


# speedup vs baseline: 1.0451x; 1.0451x over previous
import jax
import jax.numpy as jnp
from jax.experimental import pallas as pl
from jax.experimental.pallas import tpu as pltpu

_K = 1024
_D = 64
_BETA = 0.25
_TB = 2048
_BB = 2


def _vq_block(lat_ref, w_ref, q_ref, idx_ref, acc_ref, w2_ref, wbf_ref,
              wneg2_ref, iotaf_ref):
    b = pl.program_id(0)
    t = pl.program_id(1)

    @pl.when(jnp.logical_and(b == 0, t == 0))
    def _():
        w0 = w_ref[...]
        w2_ref[...] = jnp.sum(w0 * w0, axis=1)[:, None]
        wbf_ref[...] = w0.astype(jnp.bfloat16)
        wneg2_ref[...] = w0 * -2.0
        acc_ref[...] = jnp.zeros((1, _TB), jnp.float32)
        iotaf_ref[...] = jax.lax.broadcasted_iota(
            jnp.int32, (_K, 1), 0).astype(jnp.float32)

    for i in range(_BB):
        x = lat_ref[i]

        x2 = jnp.sum(x * x, axis=0, keepdims=True)
        sn2 = jax.lax.dot_general(
            wneg2_ref[...], x, (((1,), (0,)), ((), ())),
            preferred_element_type=jnp.float32,
            precision=jax.lax.Precision.DEFAULT)
        dist = (x2 + w2_ref[...]) + sn2

        m = jnp.min(dist, axis=0, keepdims=True)
        cand = jnp.where(dist == m, iotaf_ref[...], jnp.float32(_K))
        idx_f = jnp.min(cand, axis=0, keepdims=True)
        idx = idx_f.astype(jnp.int32)
        idx_ref[i, 0, :] = idx[0, :]

        iota = jax.lax.broadcasted_iota(jnp.int32, (_K, _TB), 0)
        onehot = (iota == idx).astype(jnp.bfloat16)
        q = jax.lax.dot_general(
            wbf_ref[...], onehot, (((0,), (0,)), ((), ())),
            preferred_element_type=jnp.float32)
        q_ref[i] = q

        acc_ref[...] += m


def kernel(latents, weight):
    B, D, T = latents.shape
    nt = T // _TB
    q, idx3, acc = pl.pallas_call(
        _vq_block,
        grid=(B // _BB, nt),
        in_specs=[
            pl.BlockSpec((_BB, D, _TB), lambda b, t: (b, 0, t)),
            pl.BlockSpec((_K, _D), lambda b, t: (0, 0)),
        ],
        out_specs=[
            pl.BlockSpec((_BB, D, _TB), lambda b, t: (b, 0, t)),
            pl.BlockSpec((_BB, 1, _TB), lambda b, t: (b, 0, t)),
            pl.BlockSpec((1, _TB), lambda b, t: (0, 0)),
        ],
        out_shape=[
            jax.ShapeDtypeStruct((B, D, T), jnp.float32),
            jax.ShapeDtypeStruct((B, 1, T), jnp.int32),
            jax.ShapeDtypeStruct((1, _TB), jnp.float32),
        ],
        scratch_shapes=[
            pltpu.VMEM((_K, 1), jnp.float32),
            pltpu.VMEM((_K, _D), jnp.bfloat16),
            pltpu.VMEM((_K, _D), jnp.float32),
            pltpu.VMEM((_K, 1), jnp.float32),
        ],
    )(latents, weight)
    mean_sq = jnp.sum(acc) / (B * T * D)
    loss = mean_sq + _BETA * mean_sq
    return q, loss, idx3.reshape(B, T)
